# baseline (speedup 1.0000x reference)
import functools

import jax
import jax.numpy as jnp
from jax import lax
from jax.experimental import pallas as pl
from jax.experimental.pallas import tpu as pltpu
from jax.experimental.pallas import tpu_sc as plsc

D = 64
B = 16384

_ROWS_PER_BLOCK = 2048


def _sc_gather(table, idx1, idx2):
    info = plsc.get_sparse_core_info()
    nc, ns = info.num_cores, info.num_subcores
    nw = nc * ns
    b_per_w = B // nw
    mesh = plsc.VectorSubcoreMesh(core_axis_name="c", subcore_axis_name="s")

    @functools.partial(
        pl.kernel,
        mesh=mesh,
        compiler_params=pltpu.CompilerParams(use_tc_tiling_on_sc=True),
        out_type=(
            jax.ShapeDtypeStruct((nw, b_per_w, 2 * D), jnp.float32),
            jax.ShapeDtypeStruct((nw, b_per_w, 2 * D), jnp.float32),
        ),
        scratch_types=[
            pltpu.VMEM((b_per_w,), jnp.int32),
            pltpu.VMEM((b_per_w,), jnp.int32),
            pltpu.VMEM((b_per_w, 2 * D), jnp.float32),
            pltpu.SemaphoreType.DMA,
        ],
    )
    def gather_kernel(t, i1, i2, o1, o2, iv1, iv2, rows, sem):
        wid = lax.axis_index("s") * nc + lax.axis_index("c")
        base = wid * b_per_w
        pltpu.sync_copy(i1.at[pl.ds(base, b_per_w)], iv1)
        pltpu.sync_copy(i2.at[pl.ds(base, b_per_w)], iv2)
        pltpu.async_copy(t.at[iv1], rows, sem).wait()
        pltpu.sync_copy(rows, o1.at[wid])
        pltpu.async_copy(t.at[iv2], rows, sem).wait()
        pltpu.sync_copy(rows, o2.at[wid])

    return gather_kernel(table, idx1, idx2)


_V = 100000
_FUSE_ROWS = 2000


def _fuse_body(s1_ref, s2_ref, out_ref):
    out_ref[:, :D] = s1_ref[...]
    out_ref[:, D:] = s2_ref[...]


def _fuse_tables(s1, s2):
    r = _FUSE_ROWS
    g = _V // r
    return pl.pallas_call(
        _fuse_body,
        grid=(g,),
        in_specs=[
            pl.BlockSpec((r, D), lambda i: (i, 0)),
            pl.BlockSpec((r, D), lambda i: (i, 0)),
        ],
        out_specs=pl.BlockSpec((r, 2 * D), lambda i: (i, 0)),
        out_shape=jax.ShapeDtypeStruct((_V, 2 * D), jnp.float32),
    )(s1, s2)


def _bind_cosine_body(g1_ref, g2_ref, c_ref, out_ref):
    c = c_ref[...]
    e1 = g1_ref[:, :D]
    e2 = g2_ref[:, D:]
    w1 = jnp.dot(e1, c, preferred_element_type=jnp.float32)
    w2 = jnp.dot(e2, c, preferred_element_type=jnp.float32)
    num = jnp.sum(w1 * w2, axis=-1)
    n1 = jnp.sum(w1 * w1, axis=-1)
    n2 = jnp.sum(w2 * w2, axis=-1)
    out_ref[...] = num / (jnp.sqrt(n1) * jnp.sqrt(n2) + 1e-8)


def _bind_cosine(g1, g2, circ, interpret=False):
    r = _ROWS_PER_BLOCK
    g = B // r
    out = pl.pallas_call(
        _bind_cosine_body,
        grid=(g,),
        in_specs=[
            pl.BlockSpec((r, 2 * D), lambda i: (i, 0)),
            pl.BlockSpec((r, 2 * D), lambda i: (i, 0)),
            pl.BlockSpec((D, D), lambda i: (0, 0)),
        ],
        out_specs=pl.BlockSpec((r,), lambda i: (i,)),
        out_shape=jax.ShapeDtypeStruct((B,), jnp.float32),
        interpret=interpret,
    )(g1, g2, circ)
    return out


def kernel(source1, source2, dummy_vector, word1, word2):
    i1 = word1.astype(jnp.int32)
    i2 = word2.astype(jnp.int32)
    table = _fuse_tables(source1, source2)
    g1, g2 = _sc_gather(table, i1, i2)
    g1 = g1.reshape(B, 2 * D)
    g2 = g2.reshape(B, 2 * D)
    shift = (jnp.arange(D)[None, :] - jnp.arange(D)[:, None]) % D
    circ = dummy_vector[shift]
    return _bind_cosine(g1, g2, circ)


# speedup vs baseline: 1.1151x; 1.1151x over previous
import functools

import jax
import jax.numpy as jnp
from jax import lax
from jax.experimental import pallas as pl
from jax.experimental.pallas import tpu as pltpu
from jax.experimental.pallas import tpu_sc as plsc

D = 64
B = 16384
_V = 100000

_ROWS_PER_BLOCK = 4096
_GCH = 256


def _sc_gather(t1p, t2p, idx1, idx2):
    info = plsc.get_sparse_core_info()
    nc, ns = info.num_cores, info.num_subcores
    nw = nc * ns
    bpw = B // nw
    mesh = plsc.VectorSubcoreMesh(core_axis_name="c", subcore_axis_name="s")

    @functools.partial(
        pl.kernel,
        mesh=mesh,
        compiler_params=pltpu.CompilerParams(use_tc_tiling_on_sc=True),
        out_type=(
            jax.ShapeDtypeStruct((B, 2 * D), jnp.float32),
            jax.ShapeDtypeStruct((B, 2 * D), jnp.float32),
        ),
        scratch_types=[
            pltpu.VMEM((bpw,), jnp.int32),
            pltpu.VMEM((bpw,), jnp.int32),
            pltpu.VMEM((_GCH, 2 * D), jnp.float32),
            pltpu.VMEM((_GCH, 2 * D), jnp.float32),
            pltpu.SemaphoreType.DMA,
            pltpu.SemaphoreType.DMA,
        ],
    )
    def gather_kernel(ta, tb, i1, i2, o1, o2, iv1, iv2, r1, r2, s1, s2):
        wid = lax.axis_index("s") * nc + lax.axis_index("c")
        base = wid * bpw
        pltpu.sync_copy(i1.at[pl.ds(base, bpw)], iv1)
        pltpu.sync_copy(i2.at[pl.ds(base, bpw)], iv2)
        for j in range(bpw // 16):
            iv1[pl.ds(j * 16, 16)] = lax.shift_right_logical(
                iv1[pl.ds(j * 16, 16)], 1
            )
            iv2[pl.ds(j * 16, 16)] = lax.shift_right_logical(
                iv2[pl.ds(j * 16, 16)], 1
            )
        for ch in range(bpw // _GCH):
            c1 = pltpu.async_copy(ta.at[iv1.at[pl.ds(ch * _GCH, _GCH)]], r1, s1)
            c2 = pltpu.async_copy(tb.at[iv2.at[pl.ds(ch * _GCH, _GCH)]], r2, s2)
            c1.wait()
            pltpu.sync_copy(r1, o1.at[pl.ds(base + ch * _GCH, _GCH)])
            c2.wait()
            pltpu.sync_copy(r2, o2.at[pl.ds(base + ch * _GCH, _GCH)])

    return gather_kernel(t1p, t2p, idx1, idx2)


def _bind_cosine_body(g1_ref, g2_ref, i1_ref, i2_ref, c_ref, out_ref):
    c = c_ref[...]
    par1 = jnp.bitwise_and(i1_ref[...], 1) == 1
    par2 = jnp.bitwise_and(i2_ref[...], 1) == 1
    g1 = g1_ref[...]
    g2 = g2_ref[...]
    e1 = jnp.where(par1, g1[:, D:], g1[:, :D])
    e2 = jnp.where(par2, g2[:, D:], g2[:, :D])
    w1 = jnp.dot(e1, c, preferred_element_type=jnp.float32)
    w2 = jnp.dot(e2, c, preferred_element_type=jnp.float32)
    num = jnp.sum(w1 * w2, axis=-1)
    n1 = jnp.sum(w1 * w1, axis=-1)
    n2 = jnp.sum(w2 * w2, axis=-1)
    out_ref[...] = num / (jnp.sqrt(n1) * jnp.sqrt(n2) + 1e-8)


def _bind_cosine(g1, g2, i1, i2, circ, interpret=False):
    r = _ROWS_PER_BLOCK
    g = B // r
    out = pl.pallas_call(
        _bind_cosine_body,
        grid=(g,),
        in_specs=[
            pl.BlockSpec((r, 2 * D), lambda i: (i, 0)),
            pl.BlockSpec((r, 2 * D), lambda i: (i, 0)),
            pl.BlockSpec((r, 1), lambda i: (i, 0)),
            pl.BlockSpec((r, 1), lambda i: (i, 0)),
            pl.BlockSpec((D, D), lambda i: (0, 0)),
        ],
        out_specs=pl.BlockSpec((r,), lambda i: (i,)),
        out_shape=jax.ShapeDtypeStruct((B,), jnp.float32),
        interpret=interpret,
    )(g1, g2, i1.reshape(B, 1), i2.reshape(B, 1), circ)
    return out


def _circulant(d):
    dd = jnp.concatenate([d, d])
    return jnp.stack([lax.slice(dd, (D - j,), (2 * D - j,)) for j in range(D)])


def kernel(source1, source2, dummy_vector, word1, word2):
    i1 = word1.astype(jnp.int32)
    i2 = word2.astype(jnp.int32)
    t1p = source1.reshape(_V // 2, 2 * D)
    t2p = source2.reshape(_V // 2, 2 * D)
    g1, g2 = _sc_gather(t1p, t2p, i1, i2)
    return _bind_cosine(g1, g2, i1, i2, _circulant(dummy_vector))


# speedup vs baseline: 1.3963x; 1.2522x over previous
import functools

import jax
import jax.numpy as jnp
from jax import lax
from jax.experimental import pallas as pl
from jax.experimental.pallas import tpu as pltpu
from jax.experimental.pallas import tpu_sc as plsc

D = 64
B = 16384
_V = 100000

_ROWS_PER_BLOCK = 2048
_GCH = 256


def _sc_gather(table, idx1, idx2):
    info = plsc.get_sparse_core_info()
    nc, ns = info.num_cores, info.num_subcores
    nw = nc * ns
    bpw = B // nw
    mesh = plsc.VectorSubcoreMesh(core_axis_name="c", subcore_axis_name="s")

    @functools.partial(
        pl.kernel,
        mesh=mesh,
        compiler_params=pltpu.CompilerParams(use_tc_tiling_on_sc=True),
        out_type=(
            jax.ShapeDtypeStruct((B, 2 * D), jnp.float32),
            jax.ShapeDtypeStruct((B, 2 * D), jnp.float32),
        ),
        scratch_types=[
            pltpu.VMEM((bpw,), jnp.int32),
            pltpu.VMEM((bpw,), jnp.int32),
            pltpu.VMEM((_GCH, 2 * D), jnp.float32),
            pltpu.VMEM((_GCH, 2 * D), jnp.float32),
            pltpu.SemaphoreType.DMA,
            pltpu.SemaphoreType.DMA,
        ],
    )
    def gather_kernel(t, i1, i2, o1, o2, iv1, iv2, r1, r2, s1, s2):
        wid = lax.axis_index("s") * nc + lax.axis_index("c")
        base = wid * bpw
        pltpu.sync_copy(i1.at[pl.ds(base, bpw)], iv1)
        pltpu.sync_copy(i2.at[pl.ds(base, bpw)], iv2)
        for ch in range(bpw // _GCH):
            c1 = pltpu.async_copy(t.at[iv1.at[pl.ds(ch * _GCH, _GCH)]], r1, s1)
            c2 = pltpu.async_copy(t.at[iv2.at[pl.ds(ch * _GCH, _GCH)]], r2, s2)
            c1.wait()
            pltpu.sync_copy(r1, o1.at[pl.ds(base + ch * _GCH, _GCH)])
            c2.wait()
            pltpu.sync_copy(r2, o2.at[pl.ds(base + ch * _GCH, _GCH)])

    return gather_kernel(table, idx1, idx2)


def _bind_cosine_body(g1_ref, g2_ref, c_ref, out_ref):
    c = c_ref[...]
    e1 = g1_ref[:, :D]
    e2 = g2_ref[:, D:]
    w1 = jnp.dot(e1, c, preferred_element_type=jnp.float32)
    w2 = jnp.dot(e2, c, preferred_element_type=jnp.float32)
    num = jnp.sum(w1 * w2, axis=-1)
    n1 = jnp.sum(w1 * w1, axis=-1)
    n2 = jnp.sum(w2 * w2, axis=-1)
    out_ref[...] = num / (jnp.sqrt(n1) * jnp.sqrt(n2) + 1e-8)


def _bind_cosine(g1, g2, circ, interpret=False):
    r = _ROWS_PER_BLOCK
    g = B // r
    out = pl.pallas_call(
        _bind_cosine_body,
        grid=(g,),
        in_specs=[
            pl.BlockSpec((r, 2 * D), lambda i: (i, 0)),
            pl.BlockSpec((r, 2 * D), lambda i: (i, 0)),
            pl.BlockSpec((D, D), lambda i: (0, 0)),
        ],
        out_specs=pl.BlockSpec((r,), lambda i: (i,)),
        out_shape=jax.ShapeDtypeStruct((B,), jnp.float32),
        interpret=interpret,
    )(g1, g2, circ)
    return out


def _circulant(d):
    dd = jnp.concatenate([d, d])
    return jnp.stack([lax.slice(dd, (D - j,), (2 * D - j,)) for j in range(D)])


def kernel(source1, source2, dummy_vector, word1, word2):
    i1 = word1.astype(jnp.int32)
    i2 = word2.astype(jnp.int32)
    table = jnp.concatenate([source1, source2], axis=1)
    g1, g2 = _sc_gather(table, i1, i2)
    return _bind_cosine(g1, g2, _circulant(dummy_vector))


# speedup vs baseline: 1.7412x; 1.2470x over previous
import functools

import jax
import jax.numpy as jnp
from jax import lax
from jax.experimental import pallas as pl
from jax.experimental.pallas import tpu as pltpu
from jax.experimental.pallas import tpu_sc as plsc

D = 64
B = 16384
_V = 100000

_ROWS_PER_BLOCK = 2048
_GCH = 256


def _sc_gather(table, idx1, idx2):
    info = plsc.get_sparse_core_info()
    nc, ns = info.num_cores, info.num_subcores
    nw = nc * ns
    bpw = B // nw
    mesh = plsc.VectorSubcoreMesh(core_axis_name="c", subcore_axis_name="s")

    @functools.partial(
        pl.kernel,
        mesh=mesh,
        compiler_params=pltpu.CompilerParams(use_tc_tiling_on_sc=True),
        out_type=(
            jax.ShapeDtypeStruct((B, 2 * D), jnp.float32),
            jax.ShapeDtypeStruct((B, 2 * D), jnp.float32),
        ),
        scratch_types=[
            pltpu.VMEM((bpw,), jnp.int32),
            pltpu.VMEM((bpw,), jnp.int32),
            pltpu.VMEM((_GCH, 2 * D), jnp.float32),
            pltpu.VMEM((_GCH, 2 * D), jnp.float32),
            pltpu.SemaphoreType.DMA,
            pltpu.SemaphoreType.DMA,
        ],
    )
    def gather_kernel(t, i1, i2, o1, o2, iv1, iv2, r1, r2, s1, s2):
        wid = lax.axis_index("s") * nc + lax.axis_index("c")
        base = wid * bpw
        pltpu.sync_copy(i1.at[pl.ds(base, bpw)], iv1)
        pltpu.sync_copy(i2.at[pl.ds(base, bpw)], iv2)
        for ch in range(bpw // _GCH):
            c1 = pltpu.async_copy(t.at[iv1.at[pl.ds(ch * _GCH, _GCH)]], r1, s1)
            c2 = pltpu.async_copy(t.at[iv2.at[pl.ds(ch * _GCH, _GCH)]], r2, s2)
            c1.wait()
            pltpu.sync_copy(r1, o1.at[pl.ds(base + ch * _GCH, _GCH)])
            c2.wait()
            pltpu.sync_copy(r2, o2.at[pl.ds(base + ch * _GCH, _GCH)])

    return gather_kernel(table, idx1, idx2)


def _bind_cosine_body(g1_ref, g2_ref, c_ref, out_ref):
    c = c_ref[...]
    e1 = g1_ref[:, :D]
    e2 = g2_ref[:, D:]
    w1 = jnp.dot(e1, c, preferred_element_type=jnp.float32)
    w2 = jnp.dot(e2, c, preferred_element_type=jnp.float32)
    num = jnp.sum(w1 * w2, axis=-1)
    n1 = jnp.sum(w1 * w1, axis=-1)
    n2 = jnp.sum(w2 * w2, axis=-1)
    out_ref[...] = num / (jnp.sqrt(n1) * jnp.sqrt(n2) + 1e-8)


def _bind_cosine(g1, g2, circ, interpret=False):
    r = _ROWS_PER_BLOCK
    g = B // r
    out = pl.pallas_call(
        _bind_cosine_body,
        grid=(g,),
        in_specs=[
            pl.BlockSpec((r, 2 * D), lambda i: (i, 0)),
            pl.BlockSpec((r, 2 * D), lambda i: (i, 0)),
            pl.BlockSpec((D, D), lambda i: (0, 0)),
        ],
        out_specs=pl.BlockSpec((r,), lambda i: (i,)),
        out_shape=jax.ShapeDtypeStruct((B,), jnp.float32),
        interpret=interpret,
    )(g1, g2, circ)
    return out


def _circulant(d):
    dd = jnp.concatenate([d, d])
    return jnp.stack([lax.slice(dd, (D - j,), (2 * D - j,)) for j in range(D)])


def kernel(source1, source2, dummy_vector, word1, word2):
    i1 = word1.astype(jnp.int32)
    i2 = word2.astype(jnp.int32)
    eye = jnp.eye(D, dtype=jnp.float32)
    zero = jnp.zeros((D, D), jnp.float32)
    p1 = jnp.concatenate([eye, zero], axis=1)
    p2 = jnp.concatenate([zero, eye], axis=1)
    table = jnp.dot(source1, p1) + jnp.dot(source2, p2)
    g1, g2 = _sc_gather(table, i1, i2)
    return _bind_cosine(g1, g2, _circulant(dummy_vector))


# speedup vs baseline: 2.0496x; 1.1771x over previous
import functools

import jax
import jax.numpy as jnp
from jax import lax
from jax.experimental import pallas as pl
from jax.experimental.pallas import tpu as pltpu
from jax.experimental.pallas import tpu_sc as plsc

D = 64
B = 16384
_V = 100000

_ROWS_PER_BLOCK = 2048
_GCH = 256


def _sc_gather(table1, table2, idx1, idx2):
    info = plsc.get_sparse_core_info()
    nc, ns = info.num_cores, info.num_subcores
    nw = nc * ns
    bpw = B // nw
    mesh = plsc.VectorSubcoreMesh(core_axis_name="c", subcore_axis_name="s")

    @functools.partial(
        pl.kernel,
        mesh=mesh,
        compiler_params=pltpu.CompilerParams(use_tc_tiling_on_sc=True),
        out_type=(
            jax.ShapeDtypeStruct((B, 2 * D), jnp.float32),
            jax.ShapeDtypeStruct((B, 2 * D), jnp.float32),
        ),
        scratch_types=[
            pltpu.VMEM((bpw,), jnp.int32),
            pltpu.VMEM((bpw,), jnp.int32),
            pltpu.VMEM((_GCH, 2 * D), jnp.float32),
            pltpu.VMEM((_GCH, 2 * D), jnp.float32),
            pltpu.SemaphoreType.DMA,
            pltpu.SemaphoreType.DMA,
        ],
    )
    def gather_kernel(ta, tb, i1, i2, o1, o2, iv1, iv2, r1, r2, s1, s2):
        wid = lax.axis_index("s") * nc + lax.axis_index("c")
        base = wid * bpw
        pltpu.sync_copy(i1.at[pl.ds(base, bpw)], iv1)
        pltpu.sync_copy(i2.at[pl.ds(base, bpw)], iv2)
        for ch in range(bpw // _GCH):
            c1 = pltpu.async_copy(ta.at[iv1.at[pl.ds(ch * _GCH, _GCH)]], r1, s1)
            c2 = pltpu.async_copy(tb.at[iv2.at[pl.ds(ch * _GCH, _GCH)]], r2, s2)
            c1.wait()
            pltpu.sync_copy(r1, o1.at[pl.ds(base + ch * _GCH, _GCH)])
            c2.wait()
            pltpu.sync_copy(r2, o2.at[pl.ds(base + ch * _GCH, _GCH)])

    return gather_kernel(table1, table2, idx1, idx2)


def _bind_cosine_body(g1_ref, g2_ref, c_ref, out_ref):
    c = c_ref[...]
    e1 = g1_ref[:, :D]
    e2 = g2_ref[:, :D]
    w1 = jnp.dot(e1, c, preferred_element_type=jnp.float32)
    w2 = jnp.dot(e2, c, preferred_element_type=jnp.float32)
    num = jnp.sum(w1 * w2, axis=-1)
    n1 = jnp.sum(w1 * w1, axis=-1)
    n2 = jnp.sum(w2 * w2, axis=-1)
    out_ref[...] = num / (jnp.sqrt(n1) * jnp.sqrt(n2) + 1e-8)


def _bind_cosine(g1, g2, circ, interpret=False):
    r = _ROWS_PER_BLOCK
    g = B // r
    out = pl.pallas_call(
        _bind_cosine_body,
        grid=(g,),
        in_specs=[
            pl.BlockSpec((r, 2 * D), lambda i: (i, 0)),
            pl.BlockSpec((r, 2 * D), lambda i: (i, 0)),
            pl.BlockSpec((D, D), lambda i: (0, 0)),
        ],
        out_specs=pl.BlockSpec((r,), lambda i: (i,)),
        out_shape=jax.ShapeDtypeStruct((B,), jnp.float32),
        interpret=interpret,
    )(g1, g2, circ)
    return out


def _circulant(d):
    dd = jnp.concatenate([d, d])
    return jnp.stack([lax.slice(dd, (D - j,), (2 * D - j,)) for j in range(D)])


def kernel(source1, source2, dummy_vector, word1, word2):
    i1 = word1.astype(jnp.int32)
    i2 = word2.astype(jnp.int32)
    eye = jnp.eye(D, dtype=jnp.float32)
    zero = jnp.zeros((D, D), jnp.float32)
    p = jnp.concatenate([eye, zero], axis=1)
    table1 = jnp.dot(source1, p)
    table2 = jnp.dot(source2, p)
    g1, g2 = _sc_gather(table1, table2, i1, i2)
    return _bind_cosine(g1, g2, _circulant(dummy_vector))


# speedup vs baseline: 2.0657x; 1.0078x over previous
import functools

import jax
import jax.numpy as jnp
from jax import lax
from jax.experimental import pallas as pl
from jax.experimental.pallas import tpu as pltpu
from jax.experimental.pallas import tpu_sc as plsc

D = 64
B = 16384
_V = 100000

_ROWS_PER_BLOCK = 4096
_GCH = 256


def _sc_gather(table1, table2, idx1, idx2):
    info = plsc.get_sparse_core_info()
    nc, ns = info.num_cores, info.num_subcores
    nw = nc * ns
    bpw = B // nw
    mesh = plsc.VectorSubcoreMesh(core_axis_name="c", subcore_axis_name="s")

    @functools.partial(
        pl.kernel,
        mesh=mesh,
        compiler_params=pltpu.CompilerParams(use_tc_tiling_on_sc=True),
        out_type=(
            jax.ShapeDtypeStruct((B, 2 * D), jnp.float32),
            jax.ShapeDtypeStruct((B, 2 * D), jnp.float32),
        ),
        scratch_types=[
            pltpu.VMEM((bpw,), jnp.int32),
            pltpu.VMEM((bpw,), jnp.int32),
            pltpu.VMEM((_GCH, 2 * D), jnp.float32),
            pltpu.VMEM((_GCH, 2 * D), jnp.float32),
            pltpu.SemaphoreType.DMA,
            pltpu.SemaphoreType.DMA,
        ],
    )
    def gather_kernel(ta, tb, i1, i2, o1, o2, iv1, iv2, r1, r2, s1, s2):
        wid = lax.axis_index("s") * nc + lax.axis_index("c")
        base = wid * bpw
        pltpu.sync_copy(i1.at[pl.ds(base, bpw)], iv1)
        pltpu.sync_copy(i2.at[pl.ds(base, bpw)], iv2)
        for ch in range(bpw // _GCH):
            c1 = pltpu.async_copy(ta.at[iv1.at[pl.ds(ch * _GCH, _GCH)]], r1, s1)
            c2 = pltpu.async_copy(tb.at[iv2.at[pl.ds(ch * _GCH, _GCH)]], r2, s2)
            c1.wait()
            pltpu.sync_copy(r1, o1.at[pl.ds(base + ch * _GCH, _GCH)])
            c2.wait()
            pltpu.sync_copy(r2, o2.at[pl.ds(base + ch * _GCH, _GCH)])

    return gather_kernel(table1, table2, idx1, idx2)


def _bind_cosine_body(g1_ref, g2_ref, c_ref, out_ref):
    c = c_ref[...]
    e1 = g1_ref[:, :D]
    e2 = g2_ref[:, :D]
    w1 = jnp.dot(e1, c, preferred_element_type=jnp.float32)
    w2 = jnp.dot(e2, c, preferred_element_type=jnp.float32)
    num = jnp.sum(w1 * w2, axis=-1)
    n1 = jnp.sum(w1 * w1, axis=-1)
    n2 = jnp.sum(w2 * w2, axis=-1)
    out_ref[...] = num / (jnp.sqrt(n1) * jnp.sqrt(n2) + 1e-8)


def _bind_cosine(g1, g2, circ, interpret=False):
    r = _ROWS_PER_BLOCK
    g = B // r
    out = pl.pallas_call(
        _bind_cosine_body,
        grid=(g,),
        in_specs=[
            pl.BlockSpec((r, 2 * D), lambda i: (i, 0)),
            pl.BlockSpec((r, 2 * D), lambda i: (i, 0)),
            pl.BlockSpec((D, D), lambda i: (0, 0)),
        ],
        out_specs=pl.BlockSpec((r,), lambda i: (i,)),
        out_shape=jax.ShapeDtypeStruct((B,), jnp.float32),
        interpret=interpret,
    )(g1, g2, circ)
    return out


def _circulant(d):
    dd = jnp.concatenate([d, d])
    return jnp.stack([lax.slice(dd, (D - j,), (2 * D - j,)) for j in range(D)])


def kernel(source1, source2, dummy_vector, word1, word2):
    i1 = word1.astype(jnp.int32)
    i2 = word2.astype(jnp.int32)
    eye = jnp.eye(D, dtype=jnp.float32)
    zero = jnp.zeros((D, D), jnp.float32)
    p = jnp.concatenate([eye, zero], axis=1)
    table1 = jnp.dot(source1, p)
    table2 = jnp.dot(source2, p)
    g1, g2 = _sc_gather(table1, table2, i1, i2)
    return _bind_cosine(g1, g2, _circulant(dummy_vector))


# speedup vs baseline: 2.1405x; 1.0362x over previous
import functools

import jax
import jax.numpy as jnp
from jax import lax
from jax.experimental import pallas as pl
from jax.experimental.pallas import tpu as pltpu
from jax.experimental.pallas import tpu_sc as plsc

D = 64
B = 16384
_V = 100000

_ROWS_PER_BLOCK = 4096
_GCH = 256


def _sc_gather_one(table, idx):
    info = plsc.get_sparse_core_info()
    nc, ns = info.num_cores, info.num_subcores
    nw = nc * ns
    bpw = B // nw
    nch = bpw // _GCH
    mesh = plsc.VectorSubcoreMesh(core_axis_name="c", subcore_axis_name="s")

    @functools.partial(
        pl.kernel,
        mesh=mesh,
        compiler_params=pltpu.CompilerParams(use_tc_tiling_on_sc=True),
        out_type=jax.ShapeDtypeStruct((B, 2 * D), jnp.float32),
        scratch_types=[
            pltpu.VMEM((bpw,), jnp.int32),
            pltpu.VMEM((_GCH, 2 * D), jnp.float32),
            pltpu.VMEM((_GCH, 2 * D), jnp.float32),
            pltpu.SemaphoreType.DMA,
            pltpu.SemaphoreType.DMA,
        ],
    )
    def gather_kernel(t, i, o, iv, r0, r1, s0, s1):
        wid = lax.axis_index("s") * nc + lax.axis_index("c")
        base = wid * bpw
        bufs = (r0, r1)
        sems = (s0, s1)
        pltpu.sync_copy(i.at[pl.ds(base, bpw)], iv)
        copies = [None] * nch
        copies[0] = pltpu.async_copy(t.at[iv.at[pl.ds(0, _GCH)]], r0, s0)
        for ch in range(nch):
            if ch + 1 < nch:
                copies[ch + 1] = pltpu.async_copy(
                    t.at[iv.at[pl.ds((ch + 1) * _GCH, _GCH)]],
                    bufs[(ch + 1) % 2],
                    sems[(ch + 1) % 2],
                )
            copies[ch].wait()
            pltpu.sync_copy(bufs[ch % 2], o.at[pl.ds(base + ch * _GCH, _GCH)])

    return gather_kernel(table, idx)


def _bind_cosine_body(g1_ref, g2_ref, c_ref, out_ref):
    c = c_ref[...]
    e1 = g1_ref[:, :D]
    e2 = g2_ref[:, :D]
    w1 = jnp.dot(e1, c, preferred_element_type=jnp.float32)
    w2 = jnp.dot(e2, c, preferred_element_type=jnp.float32)
    num = jnp.sum(w1 * w2, axis=-1)
    n1 = jnp.sum(w1 * w1, axis=-1)
    n2 = jnp.sum(w2 * w2, axis=-1)
    out_ref[...] = num / (jnp.sqrt(n1) * jnp.sqrt(n2) + 1e-8)


def _bind_cosine(g1, g2, circ, interpret=False):
    r = _ROWS_PER_BLOCK
    g = B // r
    out = pl.pallas_call(
        _bind_cosine_body,
        grid=(g,),
        in_specs=[
            pl.BlockSpec((r, 2 * D), lambda i: (i, 0)),
            pl.BlockSpec((r, 2 * D), lambda i: (i, 0)),
            pl.BlockSpec((D, D), lambda i: (0, 0)),
        ],
        out_specs=pl.BlockSpec((r,), lambda i: (i,)),
        out_shape=jax.ShapeDtypeStruct((B,), jnp.float32),
        interpret=interpret,
    )(g1, g2, circ)
    return out


def _circulant(d):
    dd = jnp.concatenate([d, d])
    return jnp.stack([lax.slice(dd, (D - j,), (2 * D - j,)) for j in range(D)])


def kernel(source1, source2, dummy_vector, word1, word2):
    i1 = word1.astype(jnp.int32)
    i2 = word2.astype(jnp.int32)
    eye = jnp.eye(D, dtype=jnp.float32)
    zero = jnp.zeros((D, D), jnp.float32)
    p = jnp.concatenate([eye, zero], axis=1)
    table1 = jnp.dot(source1, p)
    g1 = _sc_gather_one(table1, i1)
    table2 = jnp.dot(source2, p)
    g2 = _sc_gather_one(table2, i2)
    return _bind_cosine(g1, g2, _circulant(dummy_vector))
